# Initial kernel scaffold; baseline (speedup 1.0000x reference)
import jax
import jax.numpy as jnp
from jax.experimental import pallas as pl


def kernel(node_ids, edge_index, edge_type, embedding, weight1, weight2, gamma1, beta1, gamma2, beta2):
    raise NotImplementedError("write your pallas kernel here")


# speedup vs baseline: 2.5944x; 2.5944x over previous
import functools

import jax
import jax.numpy as jnp
from jax import lax
from jax.experimental import pallas as pl
from jax.experimental.pallas import tpu as pltpu
from jax.experimental.pallas import tpu_sc as plsc

N = 50000
R = 4
D = 64
H = 128
E = 800000

BLK = 128
TILES = 16
BLOCKS_PER_TILE = 400
EP = TILES * BLOCKS_PER_TILE * BLK
CHUNK = BLOCKS_PER_TILE * BLK

W = 25000
ACC_ROWS = 25088
ZROWS = ACC_ROWS // TILES
DUMP = 25032
COPY_TILES = 5
COPY_ROWS = W // COPY_TILES


def _make_sc_pass(table_rows: int, num_windows: int):
    num_rounds = num_windows // 2
    out_rows = num_windows * W
    mesh = plsc.VectorSubcoreMesh(core_axis_name="c", subcore_axis_name="s")

    @functools.partial(
        pl.kernel,
        out_type=jax.ShapeDtypeStruct((out_rows, D), jnp.float32),
        mesh=mesh,
        scratch_types=[
            pltpu.VMEM((BLK,), jnp.int32),
            pltpu.VMEM((BLK,), jnp.int32),
            pltpu.VMEM((BLK,), jnp.int32),
            pltpu.VMEM((BLK, D), jnp.float32),
            pltpu.VMEM_SHARED((ACC_ROWS, D), jnp.float32),
            pltpu.SemaphoreType.DMA,
            pltpu.SemaphoreType.DMA,
        ],
        compiler_params=pltpu.CompilerParams(use_tc_tiling_on_sc=False),
    )
    def sc_pass(table, iidx, oidx, zeros, out, ib, ob, lb, rows, acc, g_sem, s_sem):
        c = lax.axis_index("c")
        s = lax.axis_index("s")
        ebase = s * CHUNK
        dump16 = jnp.full((16,), DUMP, jnp.int32)
        for r in range(num_rounds):
            lo = (2 * r + c) * W
            lo16 = jnp.full((16,), lo, jnp.int32)
            hi16 = lo16 + W
            pltpu.sync_copy(zeros, acc.at[pl.ds(s * ZROWS, ZROWS)])
            plsc.subcore_barrier()

            @pl.loop(0, BLOCKS_PER_TILE)
            def _block(b):
                e0 = pl.multiple_of(ebase + b * BLK, BLK)
                pltpu.sync_copy(iidx.at[pl.ds(e0, BLK)], ib)
                pltpu.sync_copy(oidx.at[pl.ds(e0, BLK)], ob)
                for k in range(BLK // 16):
                    v = ob[pl.ds(k * 16, 16)]
                    inw = (v >= lo16) & (v < hi16)
                    lb[pl.ds(k * 16, 16)] = jnp.where(inw, v - lo16, dump16)
                pltpu.async_copy(table.at[ib], rows, g_sem).wait()
                pltpu.async_copy(rows, acc.at[lb], s_sem, add=True).wait()

            plsc.subcore_barrier()

            @pl.when(s < COPY_TILES)
            def _copy_out():
                pltpu.sync_copy(
                    acc.at[pl.ds(s * COPY_ROWS, COPY_ROWS)],
                    out.at[pl.ds(lo + s * COPY_ROWS, COPY_ROWS)],
                )

            plsc.subcore_barrier()

    return sc_pass


@functools.lru_cache(maxsize=None)
def _sc_pass(table_rows: int, num_windows: int):
    return _make_sc_pass(table_rows, num_windows)


NB = 125
BN = N // NB


def _mid_body(a_ref, w1_ref, w2_ref, g1_ref, b1_ref, y_ref):
    a = a_ref[...]
    acc = jnp.dot(a[0], w1_ref[0], preferred_element_type=jnp.float32)
    for r in range(1, R):
        acc += jnp.dot(a[r], w1_ref[r], preferred_element_type=jnp.float32)
    mu = jnp.mean(acc, axis=-1, keepdims=True)
    dev = acc - mu
    var = jnp.mean(dev * dev, axis=-1, keepdims=True)
    x1 = dev * lax.rsqrt(var + 1e-5) * g1_ref[...] + b1_ref[...]
    x1 = jnp.maximum(x1, 0.0)
    for r in range(R):
        y_ref[r] = jnp.dot(x1, w2_ref[r], preferred_element_type=jnp.float32)


def _tc_mid(a, w1, w2, g1, b1):
    return pl.pallas_call(
        _mid_body,
        grid=(NB,),
        in_specs=[
            pl.BlockSpec((R, BN, D), lambda i: (0, i, 0)),
            pl.BlockSpec((R, D, H), lambda i: (0, 0, 0)),
            pl.BlockSpec((R, H, D), lambda i: (0, 0, 0)),
            pl.BlockSpec((1, H), lambda i: (0, 0)),
            pl.BlockSpec((1, H), lambda i: (0, 0)),
        ],
        out_specs=pl.BlockSpec((R, BN, D), lambda i: (0, i, 0)),
        out_shape=jax.ShapeDtypeStruct((R, N, D), jnp.float32),
    )(a, w1, w2, g1, b1)


def _final_body(acc_ref, x_ref, g2_ref, b2_ref, o_ref):
    s = acc_ref[...]
    mu = jnp.mean(s, axis=-1, keepdims=True)
    dev = s - mu
    var = jnp.mean(dev * dev, axis=-1, keepdims=True)
    o_ref[...] = dev * lax.rsqrt(var + 1e-5) * g2_ref[...] + b2_ref[...] + x_ref[...]


def _tc_final(acc2, x, g2, b2):
    return pl.pallas_call(
        _final_body,
        grid=(NB,),
        in_specs=[
            pl.BlockSpec((BN, D), lambda i: (i, 0)),
            pl.BlockSpec((BN, D), lambda i: (i, 0)),
            pl.BlockSpec((1, D), lambda i: (0, 0)),
            pl.BlockSpec((1, D), lambda i: (0, 0)),
        ],
        out_specs=pl.BlockSpec((BN, D), lambda i: (i, 0)),
        out_shape=jax.ShapeDtypeStruct((N, D), jnp.float32),
    )(acc2, x, g2, b2)


def kernel(node_ids, edge_index, edge_type, embedding, weight1, weight2,
           gamma1, beta1, gamma2, beta2):
    src = edge_index[0].astype(jnp.int32)
    dst = edge_index[1].astype(jnp.int32)
    et = edge_type.astype(jnp.int32)
    x = jnp.take(embedding, node_ids.astype(jnp.int32), axis=0)

    pad = EP - E
    zero_pad = jnp.zeros((pad,), jnp.int32)
    sink1 = jnp.full((pad,), 2 * R * W, jnp.int32)
    sink2 = jnp.full((pad,), 2 * W, jnp.int32)
    iidx1 = jnp.concatenate([src, zero_pad])
    oidx1 = jnp.concatenate([et * N + dst, sink1])
    iidx2 = jnp.concatenate([et * N + src, zero_pad])
    oidx2 = jnp.concatenate([dst, zero_pad + sink2])

    zeros = jnp.zeros((ZROWS, D), jnp.float32)
    a = _sc_pass(N, 2 * R)(x, iidx1, oidx1, zeros)
    y = _tc_mid(a.reshape(R, N, D), weight1, weight2,
                gamma1.reshape(1, H), beta1.reshape(1, H))
    acc2 = _sc_pass(R * N, 2)(y.reshape(R * N, D), iidx2, oidx2, zeros)
    return _tc_final(acc2, x, gamma2.reshape(1, D), beta2.reshape(1, D))


# speedup vs baseline: 2.6030x; 1.0033x over previous
import functools

import jax
import jax.numpy as jnp
from jax import lax
from jax.experimental import pallas as pl
from jax.experimental.pallas import tpu as pltpu
from jax.experimental.pallas import tpu_sc as plsc

N = 50000
R = 4
D = 64
H = 128
E = 800000

BLK = 128
TILES = 16
BLOCKS_PER_TILE = 400
EP = TILES * BLOCKS_PER_TILE * BLK
CHUNK = BLOCKS_PER_TILE * BLK

W = 25000
ACC_ROWS = 25088
ZROWS = ACC_ROWS // TILES
DUMP = 25000
COPY_TILES = 5
COPY_ROWS = W // COPY_TILES


def _make_sc_pass(table_rows: int, num_windows: int):
    num_rounds = num_windows // 2
    out_rows = num_windows * W
    mesh = plsc.VectorSubcoreMesh(core_axis_name="c", subcore_axis_name="s")

    @functools.partial(
        pl.kernel,
        out_type=jax.ShapeDtypeStruct((out_rows, D), jnp.float32),
        mesh=mesh,
        scratch_types=[
            pltpu.VMEM((BLK,), jnp.int32),
            pltpu.VMEM((BLK,), jnp.int32),
            pltpu.VMEM((BLK,), jnp.int32),
            pltpu.VMEM((BLK, D), jnp.float32),
            pltpu.VMEM_SHARED((ACC_ROWS, D), jnp.float32),
            pltpu.SemaphoreType.DMA,
            pltpu.SemaphoreType.DMA,
        ],
        compiler_params=pltpu.CompilerParams(use_tc_tiling_on_sc=False),
    )
    def sc_pass(table, iidx, oidx, zeros, out, ib, ob, lb, rows, acc, g_sem, s_sem):
        c = lax.axis_index("c")
        s = lax.axis_index("s")
        ebase = s * CHUNK
        dump16 = jnp.full((16,), DUMP, jnp.int32)
        for r in range(num_rounds):
            lo = (2 * r + c) * W
            lo16 = jnp.full((16,), lo, jnp.int32)
            hi16 = lo16 + W
            pltpu.sync_copy(zeros, acc.at[pl.ds(s * ZROWS, ZROWS)])
            plsc.subcore_barrier()

            @pl.loop(0, BLOCKS_PER_TILE)
            def _block(b):
                e0 = pl.multiple_of(ebase + b * BLK, BLK)
                pltpu.sync_copy(iidx.at[pl.ds(e0, BLK)], ib)
                pltpu.sync_copy(oidx.at[pl.ds(e0, BLK)], ob)
                for k in range(BLK // 16):
                    v = ob[pl.ds(k * 16, 16)]
                    inw = (v >= lo16) & (v < hi16)
                    lb[pl.ds(k * 16, 16)] = jnp.where(
                        inw, v - lo16, dump16 + (v & 63))
                pltpu.async_copy(table.at[ib], rows, g_sem).wait()
                pltpu.async_copy(rows, acc.at[lb], s_sem, add=True).wait()

            plsc.subcore_barrier()

            @pl.when(s < COPY_TILES)
            def _copy_out():
                pltpu.sync_copy(
                    acc.at[pl.ds(s * COPY_ROWS, COPY_ROWS)],
                    out.at[pl.ds(lo + s * COPY_ROWS, COPY_ROWS)],
                )

            plsc.subcore_barrier()

    return sc_pass


@functools.lru_cache(maxsize=None)
def _sc_pass(table_rows: int, num_windows: int):
    return _make_sc_pass(table_rows, num_windows)


NB = 125
BN = N // NB


def _mid_body(a_ref, w1_ref, w2_ref, g1_ref, b1_ref, y_ref):
    a = a_ref[...]
    acc = jnp.dot(a[0], w1_ref[0], preferred_element_type=jnp.float32)
    for r in range(1, R):
        acc += jnp.dot(a[r], w1_ref[r], preferred_element_type=jnp.float32)
    mu = jnp.mean(acc, axis=-1, keepdims=True)
    dev = acc - mu
    var = jnp.mean(dev * dev, axis=-1, keepdims=True)
    x1 = dev * lax.rsqrt(var + 1e-5) * g1_ref[...] + b1_ref[...]
    x1 = jnp.maximum(x1, 0.0)
    for r in range(R):
        y_ref[r] = jnp.dot(x1, w2_ref[r], preferred_element_type=jnp.float32)


def _tc_mid(a, w1, w2, g1, b1):
    return pl.pallas_call(
        _mid_body,
        grid=(NB,),
        in_specs=[
            pl.BlockSpec((R, BN, D), lambda i: (0, i, 0)),
            pl.BlockSpec((R, D, H), lambda i: (0, 0, 0)),
            pl.BlockSpec((R, H, D), lambda i: (0, 0, 0)),
            pl.BlockSpec((1, H), lambda i: (0, 0)),
            pl.BlockSpec((1, H), lambda i: (0, 0)),
        ],
        out_specs=pl.BlockSpec((R, BN, D), lambda i: (0, i, 0)),
        out_shape=jax.ShapeDtypeStruct((R, N, D), jnp.float32),
    )(a, w1, w2, g1, b1)


def _final_body(acc_ref, x_ref, g2_ref, b2_ref, o_ref):
    s = acc_ref[...]
    mu = jnp.mean(s, axis=-1, keepdims=True)
    dev = s - mu
    var = jnp.mean(dev * dev, axis=-1, keepdims=True)
    o_ref[...] = dev * lax.rsqrt(var + 1e-5) * g2_ref[...] + b2_ref[...] + x_ref[...]


def _tc_final(acc2, x, g2, b2):
    return pl.pallas_call(
        _final_body,
        grid=(NB,),
        in_specs=[
            pl.BlockSpec((BN, D), lambda i: (i, 0)),
            pl.BlockSpec((BN, D), lambda i: (i, 0)),
            pl.BlockSpec((1, D), lambda i: (0, 0)),
            pl.BlockSpec((1, D), lambda i: (0, 0)),
        ],
        out_specs=pl.BlockSpec((BN, D), lambda i: (i, 0)),
        out_shape=jax.ShapeDtypeStruct((N, D), jnp.float32),
    )(acc2, x, g2, b2)


def kernel(node_ids, edge_index, edge_type, embedding, weight1, weight2,
           gamma1, beta1, gamma2, beta2):
    src = edge_index[0].astype(jnp.int32)
    dst = edge_index[1].astype(jnp.int32)
    et = edge_type.astype(jnp.int32)
    x = jnp.take(embedding, node_ids.astype(jnp.int32), axis=0)

    pad = EP - E
    zero_pad = jnp.zeros((pad,), jnp.int32)
    sink1 = jnp.full((pad,), 2 * R * W, jnp.int32)
    sink2 = jnp.full((pad,), 2 * W, jnp.int32)
    iidx1 = jnp.concatenate([src, zero_pad])
    oidx1 = jnp.concatenate([et * N + dst, sink1])
    iidx2 = jnp.concatenate([et * N + src, zero_pad])
    oidx2 = jnp.concatenate([dst, zero_pad + sink2])

    zeros = jnp.zeros((ZROWS, D), jnp.float32)
    a = _sc_pass(N, 2 * R)(x, iidx1, oidx1, zeros)
    y = _tc_mid(a.reshape(R, N, D), weight1, weight2,
                gamma1.reshape(1, H), beta1.reshape(1, H))
    acc2 = _sc_pass(R * N, 2)(y.reshape(R * N, D), iidx2, oidx2, zeros)
    return _tc_final(acc2, x, gamma2.reshape(1, D), beta2.reshape(1, D))


# speedup vs baseline: 3.5377x; 1.3591x over previous
import functools

import jax
import jax.numpy as jnp
from jax import lax
from jax.experimental import pallas as pl
from jax.experimental.pallas import tpu as pltpu
from jax.experimental.pallas import tpu_sc as plsc

N = 50000
R = 4
D = 64
H = 128
E = 800000

BLK = 64
TILES = 16
EP = 819200
CHUNK = EP // TILES
SB_BLOCKS = 16
SBE = SB_BLOCKS * BLK
NSB = CHUNK // SBE
SLOTS = 4

W = 25000
ACC_ROWS = 25088
ZROWS = ACC_ROWS // TILES
DUMP = W
COPY_TILES = 5
COPY_ROWS = W // COPY_TILES
WINDOWS1 = (R * N) // W
WINDOWS2 = N // W


def _make_sc_pass(num_windows: int):
    num_rounds = num_windows // 2
    out_rows = num_windows * W
    mesh = plsc.VectorSubcoreMesh(core_axis_name="c", subcore_axis_name="s")

    @functools.partial(
        pl.kernel,
        out_type=jax.ShapeDtypeStruct((out_rows, D), jnp.float32),
        mesh=mesh,
        scratch_types=[
            pltpu.VMEM((2, SBE), jnp.int32),
            pltpu.VMEM((2, SB_BLOCKS, BLK), jnp.int32),
            pltpu.VMEM((SLOTS, BLK, D), jnp.float32),
            pltpu.VMEM_SHARED((ACC_ROWS, D), jnp.float32),
        ] + [pltpu.SemaphoreType.DMA] * (SLOTS + 2),
        compiler_params=pltpu.CompilerParams(use_tc_tiling_on_sc=False),
    )
    def sc_pass(table, iidx, oidx, zeros, out, isb, osb, rows, acc, *sems):
        gsem = sems[:SLOTS]
        isem, osem = sems[SLOTS], sems[SLOTS + 1]
        c = lax.axis_index("c")
        s = lax.axis_index("s")
        dump16 = jnp.full((16,), DUMP, jnp.int32)

        def issue_idx(sb, q):
            e0 = pl.multiple_of(s * CHUNK + sb * SBE, SBE)
            r0 = pl.multiple_of(s * (CHUNK // BLK) + sb * SB_BLOCKS, SB_BLOCKS)
            pltpu.async_copy(iidx.at[pl.ds(e0, SBE)], isb.at[q], isem)
            pltpu.async_copy(oidx.at[pl.ds(r0, SB_BLOCKS)], osb.at[q], osem)

        def wait_idx(q):
            pltpu.make_async_copy(iidx.at[pl.ds(0, SBE)], isb.at[q], isem).wait()
            pltpu.make_async_copy(oidx.at[pl.ds(0, SB_BLOCKS)], osb.at[q],
                                  osem).wait()

        def issue_gather(q, bl, slot):
            idxsl = isb.at[q, pl.ds(bl * BLK, BLK)]
            pltpu.async_copy(table.at[idxsl], rows.at[slot], gsem[slot])

        def wait_gather(q, slot):
            pltpu.make_async_copy(
                table.at[isb.at[q, pl.ds(0, BLK)]], rows.at[slot], gsem[slot]
            ).wait()

        for r in range(num_rounds):
            lo = (2 * r + c) * W
            lo16 = jnp.full((16,), lo, jnp.int32)
            hi16 = lo16 + W
            pltpu.sync_copy(zeros, acc.at[pl.ds(s * ZROWS, ZROWS)])
            plsc.subcore_barrier()
            issue_idx(0, 0)

            @pl.loop(0, NSB)
            def _sb(sb):
                q = jnp.bitwise_and(sb, 1)
                wait_idx(q)

                @pl.when(sb + 1 < NSB)
                def _next():
                    issue_idx(sb + 1, 1 - q)

                for bl in range(SB_BLOCKS):
                    for k in range(BLK // 16):
                        v = osb[q, bl, pl.ds(k * 16, 16)]
                        inw = (v >= lo16) & (v < hi16)
                        osb[q, bl, pl.ds(k * 16, 16)] = jnp.where(
                            inw, v - lo16, dump16 + (v & 63))

                for bl in range(SLOTS):
                    issue_gather(q, bl, bl)
                for bl in range(SB_BLOCKS):
                    slot = bl % SLOTS
                    wait_gather(q, slot)
                    pltpu.sync_copy(rows.at[slot], acc.at[osb.at[q, bl]],
                                    add=True)
                    if bl + SLOTS < SB_BLOCKS:
                        issue_gather(q, bl + SLOTS, slot)

            plsc.subcore_barrier()

            @pl.when(s < COPY_TILES)
            def _copy_out():
                pltpu.sync_copy(
                    acc.at[pl.ds(s * COPY_ROWS, COPY_ROWS)],
                    out.at[pl.ds(lo + s * COPY_ROWS, COPY_ROWS)],
                )

            plsc.subcore_barrier()

    return sc_pass


@functools.lru_cache(maxsize=None)
def _sc_pass(num_windows: int):
    return _make_sc_pass(num_windows)


NB = 125
BN = N // NB


def _mid_body(a0, a1, a2, a3, w1_ref, w2_ref, g1_ref, b1_ref, y_ref):
    a = (a0, a1, a2, a3)
    acc = jnp.dot(a[0][...], w1_ref[0], preferred_element_type=jnp.float32)
    for r in range(1, R):
        acc += jnp.dot(a[r][...], w1_ref[r], preferred_element_type=jnp.float32)
    mu = jnp.mean(acc, axis=-1, keepdims=True)
    dev = acc - mu
    var = jnp.mean(dev * dev, axis=-1, keepdims=True)
    x1 = dev * lax.rsqrt(var + 1e-5) * g1_ref[...] + b1_ref[...]
    x1 = jnp.maximum(x1, 0.0)
    for r in range(R):
        y_ref[r] = jnp.dot(x1, w2_ref[r], preferred_element_type=jnp.float32)


def _tc_mid(a_flat, w1, w2, g1, b1):
    specs = [
        pl.BlockSpec((BN, D), functools.partial(lambda r, i: (r * NB + i, 0), r))
        for r in range(R)
    ]
    return pl.pallas_call(
        _mid_body,
        grid=(NB,),
        in_specs=specs + [
            pl.BlockSpec((R, D, H), lambda i: (0, 0, 0)),
            pl.BlockSpec((R, H, D), lambda i: (0, 0, 0)),
            pl.BlockSpec((1, H), lambda i: (0, 0)),
            pl.BlockSpec((1, H), lambda i: (0, 0)),
        ],
        out_specs=pl.BlockSpec((R, BN, D), lambda i: (0, i, 0)),
        out_shape=jax.ShapeDtypeStruct((R, N, D), jnp.float32),
    )(a_flat, a_flat, a_flat, a_flat, w1, w2, g1, b1)


def _final_body(acc_ref, x_ref, g2_ref, b2_ref, o_ref):
    sm = acc_ref[...]
    mu = jnp.mean(sm, axis=-1, keepdims=True)
    dev = sm - mu
    var = jnp.mean(dev * dev, axis=-1, keepdims=True)
    o_ref[...] = dev * lax.rsqrt(var + 1e-5) * g2_ref[...] + b2_ref[...] + x_ref[...]


def _tc_final(acc2, x, g2, b2):
    return pl.pallas_call(
        _final_body,
        grid=(NB,),
        in_specs=[
            pl.BlockSpec((BN, D), lambda i: (i, 0)),
            pl.BlockSpec((BN, D), lambda i: (i, 0)),
            pl.BlockSpec((1, D), lambda i: (0, 0)),
            pl.BlockSpec((1, D), lambda i: (0, 0)),
        ],
        out_specs=pl.BlockSpec((BN, D), lambda i: (i, 0)),
        out_shape=jax.ShapeDtypeStruct((N, D), jnp.float32),
    )(acc2, x, g2, b2)


def kernel(node_ids, edge_index, edge_type, embedding, weight1, weight2,
           gamma1, beta1, gamma2, beta2):
    src = edge_index[0].astype(jnp.int32)
    dst = edge_index[1].astype(jnp.int32)
    et = edge_type.astype(jnp.int32)
    x = jnp.take(embedding, node_ids.astype(jnp.int32), axis=0)

    pad = EP - E
    zero_pad = jnp.zeros((pad,), jnp.int32)
    sink1 = jnp.full((pad,), WINDOWS1 * W, jnp.int32)
    sink2 = jnp.full((pad,), WINDOWS2 * W, jnp.int32)
    iidx1 = jnp.concatenate([src, zero_pad])
    oidx1 = jnp.concatenate([et * N + dst, sink1]).reshape(EP // BLK, BLK)
    iidx2 = jnp.concatenate([et * N + src, zero_pad])
    oidx2 = jnp.concatenate([dst, sink2]).reshape(EP // BLK, BLK)

    zeros = jnp.zeros((ZROWS, D), jnp.float32)
    a = _sc_pass(WINDOWS1)(x, iidx1, oidx1, zeros)
    y = _tc_mid(a, weight1, weight2,
                gamma1.reshape(1, H), beta1.reshape(1, H))
    acc2 = _sc_pass(WINDOWS2)(y.reshape(R * N, D), iidx2, oidx2, zeros)
    return _tc_final(acc2, x, gamma2.reshape(1, D), beta2.reshape(1, D))


# speedup vs baseline: 8.7308x; 2.4679x over previous
import functools

import jax
import jax.numpy as jnp
from jax import lax
from jax.experimental import pallas as pl
from jax.experimental.pallas import tpu as pltpu
from jax.experimental.pallas import tpu_sc as plsc

N = 50000
R = 4
D = 64
H = 128
E = 800000

BLK = 64
TILES = 16
EP = 819200
CHUNK = EP // TILES
SBE = 1024
NSB = CHUNK // SBE
VECS = SBE // 16
CAP = 2048
FLUSH = 960
SLOTS = 4

W = 20000
ACC_ROWS = 20096
ZROWS = ACC_ROWS // TILES
DUMP = W
COPY_TILES = 5
COPY_ROWS = W // COPY_TILES
WINDOWS1 = (R * N) // W
WINDOWS2 = 4


def _make_sc_pass(num_windows: int):
    num_rounds = (num_windows + 1) // 2
    out_rows = num_windows * W
    mesh = plsc.VectorSubcoreMesh(core_axis_name="c", subcore_axis_name="s")

    @functools.partial(
        pl.kernel,
        out_type=jax.ShapeDtypeStruct((out_rows, D), jnp.float32),
        mesh=mesh,
        scratch_types=[
            pltpu.VMEM((2, SBE), jnp.int32),
            pltpu.VMEM((2, SBE), jnp.int32),
            pltpu.VMEM((SBE,), jnp.int32),
            pltpu.VMEM((SBE,), jnp.int32),
            pltpu.VMEM((SBE,), jnp.int32),
            pltpu.SMEM((SBE,), jnp.int32),
            pltpu.VMEM((CAP,), jnp.int32),
            pltpu.VMEM((CAP,), jnp.int32),
            pltpu.VMEM((SLOTS, BLK), jnp.int32),
            pltpu.VMEM((SLOTS, BLK, D), jnp.float32),
            pltpu.VMEM_SHARED((ACC_ROWS, D), jnp.float32),
            pltpu.VMEM_SHARED((TILES, SBE), jnp.int32),
        ] + [pltpu.SemaphoreType.DMA] * (SLOTS + 2),
        compiler_params=pltpu.CompilerParams(use_tc_tiling_on_sc=False),
    )
    def sc_pass(table, iidx, oidx, zeros, out, isb, osb, pbi, pbo, cntb,
                cnts, ci, co, cob, rows, acc, cshr, *sems):
        gsem = sems[:SLOTS]
        isem, osem = sems[SLOTS], sems[SLOTS + 1]
        c = lax.axis_index("c")
        s = lax.axis_index("s")
        iota16 = lax.iota(jnp.int32, 16)
        dump16 = jnp.full((16,), DUMP, jnp.int32)
        z16 = jnp.zeros((16,), jnp.int32)
        lane15 = jnp.full((16,), 15, jnp.int32)

        def lane_gather(vec, idx):
            return lax.gather(
                vec, idx[:, None],
                dimension_numbers=lax.GatherDimensionNumbers(
                    offset_dims=(), collapsed_slice_dims=(0,),
                    start_index_map=(0,)),
                slice_sizes=(1,),
                mode=lax.GatherScatterMode.PROMISE_IN_BOUNDS)

        def issue_idx(sb, q):
            e0 = pl.multiple_of(s * CHUNK + sb * SBE, SBE)
            pltpu.async_copy(iidx.at[pl.ds(e0, SBE)], isb.at[q], isem)
            pltpu.async_copy(oidx.at[pl.ds(e0, SBE)], osb.at[q], osem)

        def wait_idx(q):
            pltpu.make_async_copy(iidx.at[pl.ds(0, SBE)], isb.at[q], isem).wait()
            pltpu.make_async_copy(oidx.at[pl.ds(0, SBE)], osb.at[q], osem).wait()

        def issue_gather(bl, slot):
            idxsl = ci.at[pl.ds(bl * BLK, BLK)]
            pltpu.async_copy(table.at[idxsl], rows.at[slot], gsem[slot])

        def wait_gather(slot):
            pltpu.make_async_copy(
                table.at[ci.at[pl.ds(0, BLK)]], rows.at[slot], gsem[slot]
            ).wait()

        def flush(nblk, limit):
            limit16 = jnp.full((16,), limit, jnp.int32)
            for p in range(SLOTS):
                @pl.when(p < nblk)
                def _prime():
                    issue_gather(p, p)

            @pl.loop(0, lax.div(nblk + (SLOTS - 1), SLOTS))
            def _grp(jg):
                for p in range(SLOTS):
                    bl = jg * SLOTS + p

                    @pl.when(bl < nblk)
                    def _blk():
                        wait_gather(p)
                        for k in range(BLK // 16):
                            off = bl * BLK + k * 16
                            vc = co[pl.ds(off, 16)]
                            pos = jnp.full((16,), off, jnp.int32) + iota16
                            cob[p, pl.ds(k * 16, 16)] = jnp.where(
                                pos < limit16, vc, dump16)
                        pltpu.sync_copy(rows.at[p], acc.at[cob.at[p]],
                                        add=True)

                        @pl.when(bl + SLOTS < nblk)
                        def _refill():
                            issue_gather(bl + SLOTS, p)

        @pl.loop(0, CAP // 16)
        def _init(i):
            ci[pl.ds(i * 16, 16)] = z16

        for r in range(num_rounds):
            w = 2 * r + c
            lo = w * W
            lo16 = jnp.full((16,), lo, jnp.int32)
            hi16 = lo16 + W
            pltpu.sync_copy(zeros, acc.at[pl.ds(s * ZROWS, ZROWS)])
            plsc.subcore_barrier()
            issue_idx(0, 0)

            def sb_body(sb, cur):
                q = jnp.bitwise_and(sb, 1)
                wait_idx(q)

                @pl.when(sb + 1 < NSB)
                def _next():
                    issue_idx(sb + 1, 1 - q)

                @pl.loop(0, VECS)
                def _p1(v):
                    voff = v * 16
                    vi = isb[q, pl.ds(voff, 16)]
                    vo = osb[q, pl.ds(voff, 16)]
                    m = (vo >= lo16) & (vo < hi16)
                    mi = jnp.where(m, 1, 0)
                    cs = mi
                    for sh in (1, 2, 4, 8):
                        g = lane_gather(cs, jnp.maximum(iota16 - sh, 0))
                        cs = cs + jnp.where(iota16 >= sh, g, 0)
                    perm = z16
                    for step in (8, 4, 2, 1):
                        probe = lane_gather(cs, perm + (step - 1))
                        perm = jnp.where(probe <= iota16, perm + step, perm)
                    permc = jnp.minimum(perm, lane15)
                    cnt16 = lane_gather(cs, lane15)
                    pbi[pl.ds(voff, 16)] = lane_gather(vi, permc)
                    pbo[pl.ds(voff, 16)] = jnp.where(
                        iota16 < cnt16, lane_gather(vo, permc) - lo16, dump16)
                    cntb[pl.ds(voff, 16)] = cnt16

                plsc.subcore_barrier()
                pltpu.sync_copy(cntb, cshr.at[s])
                pltpu.sync_copy(cshr.at[s], cnts)

                off = jnp.int32(0)
                for v in range(VECS):
                    voff = v * 16
                    dst = jnp.minimum(cur + off, CAP - 16)
                    ci[pl.ds(dst, 16)] = pbi[pl.ds(voff, 16)]
                    co[pl.ds(dst, 16)] = pbo[pl.ds(voff, 16)]
                    off = off + jnp.minimum(jnp.maximum(cnts[voff], 0), 16)

                cur = jnp.minimum(cur + off, FLUSH + SBE)
                nblk = jnp.where(cur >= FLUSH, lax.div(cur, BLK), 0)
                flush(nblk, nblk * BLK)

                @pl.when(nblk > 0)
                def _shift_tail():
                    for k in range(BLK // 16):
                        ci[pl.ds(k * 16, 16)] = ci[pl.ds(nblk * BLK + k * 16, 16)]
                        co[pl.ds(k * 16, 16)] = co[pl.ds(nblk * BLK + k * 16, 16)]

                return cur - nblk * BLK

            cur = pl.loop(0, NSB, init_carry=jnp.int32(0))(sb_body)
            flush(lax.div(cur + (BLK - 1), BLK), cur)
            plsc.subcore_barrier()

            @pl.when((s < COPY_TILES) & (w < num_windows))
            def _copy_out():
                pltpu.sync_copy(
                    acc.at[pl.ds(s * COPY_ROWS, COPY_ROWS)],
                    out.at[pl.ds(lo + s * COPY_ROWS, COPY_ROWS)],
                )

            plsc.subcore_barrier()

    return sc_pass


@functools.lru_cache(maxsize=None)
def _sc_pass(num_windows: int):
    return _make_sc_pass(num_windows)


NB = 125
BN = N // NB


def _mid_body(a0, a1, a2, a3, w1_ref, w2_ref, g1_ref, b1_ref, y_ref):
    a = (a0, a1, a2, a3)
    acc = jnp.dot(a[0][...], w1_ref[0], preferred_element_type=jnp.float32)
    for r in range(1, R):
        acc += jnp.dot(a[r][...], w1_ref[r], preferred_element_type=jnp.float32)
    mu = jnp.mean(acc, axis=-1, keepdims=True)
    dev = acc - mu
    var = jnp.mean(dev * dev, axis=-1, keepdims=True)
    x1 = dev * lax.rsqrt(var + 1e-5) * g1_ref[...] + b1_ref[...]
    x1 = jnp.maximum(x1, 0.0)
    for r in range(R):
        y_ref[r] = jnp.dot(x1, w2_ref[r], preferred_element_type=jnp.float32)


def _tc_mid(a_flat, w1, w2, g1, b1):
    specs = [
        pl.BlockSpec((BN, D), functools.partial(lambda r, i: (r * NB + i, 0), r))
        for r in range(R)
    ]
    return pl.pallas_call(
        _mid_body,
        grid=(NB,),
        in_specs=specs + [
            pl.BlockSpec((R, D, H), lambda i: (0, 0, 0)),
            pl.BlockSpec((R, H, D), lambda i: (0, 0, 0)),
            pl.BlockSpec((1, H), lambda i: (0, 0)),
            pl.BlockSpec((1, H), lambda i: (0, 0)),
        ],
        out_specs=pl.BlockSpec((R, BN, D), lambda i: (0, i, 0)),
        out_shape=jax.ShapeDtypeStruct((R, N, D), jnp.float32),
    )(a_flat, a_flat, a_flat, a_flat, w1, w2, g1, b1)


def _final_body(acc_ref, x_ref, g2_ref, b2_ref, o_ref):
    sm = acc_ref[...]
    mu = jnp.mean(sm, axis=-1, keepdims=True)
    dev = sm - mu
    var = jnp.mean(dev * dev, axis=-1, keepdims=True)
    o_ref[...] = dev * lax.rsqrt(var + 1e-5) * g2_ref[...] + b2_ref[...] + x_ref[...]


def _tc_final(acc2, x, g2, b2):
    return pl.pallas_call(
        _final_body,
        grid=(NB,),
        in_specs=[
            pl.BlockSpec((BN, D), lambda i: (i, 0)),
            pl.BlockSpec((BN, D), lambda i: (i, 0)),
            pl.BlockSpec((1, D), lambda i: (0, 0)),
            pl.BlockSpec((1, D), lambda i: (0, 0)),
        ],
        out_specs=pl.BlockSpec((BN, D), lambda i: (i, 0)),
        out_shape=jax.ShapeDtypeStruct((N, D), jnp.float32),
    )(acc2, x, g2, b2)


def kernel(node_ids, edge_index, edge_type, embedding, weight1, weight2,
           gamma1, beta1, gamma2, beta2):
    src = edge_index[0].astype(jnp.int32)
    dst = edge_index[1].astype(jnp.int32)
    et = edge_type.astype(jnp.int32)
    x = jnp.take(embedding, node_ids.astype(jnp.int32), axis=0)

    pad = EP - E
    zero_pad = jnp.zeros((pad,), jnp.int32)
    sink1 = jnp.full((pad,), WINDOWS1 * W, jnp.int32)
    sink2 = jnp.full((pad,), WINDOWS2 * W, jnp.int32)
    iidx1 = jnp.concatenate([src, zero_pad])
    oidx1 = jnp.concatenate([et * N + dst, sink1])
    iidx2 = jnp.concatenate([et * N + src, zero_pad])
    oidx2 = jnp.concatenate([dst, sink2])

    zeros = jnp.zeros((ZROWS, D), jnp.float32)
    a = _sc_pass(WINDOWS1)(x, iidx1, oidx1, zeros)
    y = _tc_mid(a, weight1, weight2,
                gamma1.reshape(1, H), beta1.reshape(1, H))
    acc2 = _sc_pass(WINDOWS2)(y.reshape(R * N, D), iidx2, oidx2, zeros)
    return _tc_final(acc2, x, gamma2.reshape(1, D), beta2.reshape(1, D))
